# baseline (speedup 1.0000x reference)
import jax
import jax.numpy as jnp
from jax import lax
from jax.experimental import pallas as pl
from jax.experimental.pallas import tpu as pltpu
from jax.experimental.pallas import tpu_sc as plsc

N_NODES = 10000
N_EDGES = 320000
HIDDEN = 128
NUM_SC = 2
NUM_SUBCORES = 16
NUM_WORKERS = NUM_SC * NUM_SUBCORES
EDGES_PER_WORKER = N_EDGES // NUM_WORKERS
ROW_BLOCK = 1000


def _sc_count_body(tgt_hbm, zeros_hbm, ones_hbm, cnt0_hbm, cnt1_hbm,
                   idx_v, ones_v, cnt_sh):
    c = lax.axis_index("c")
    s = lax.axis_index("s")
    wid = c * NUM_SUBCORES + s

    @pl.when(s == 0)
    def _zero():
        pltpu.sync_copy(zeros_hbm, cnt_sh)

    plsc.subcore_barrier()

    pltpu.sync_copy(
        tgt_hbm.at[pl.ds(wid * EDGES_PER_WORKER, EDGES_PER_WORKER)], idx_v)
    pltpu.sync_copy(ones_hbm, ones_v)
    pltpu.sync_copy(ones_v, cnt_sh.at[idx_v], add=True)

    plsc.subcore_barrier()

    @pl.when((s == 0) & (c == 0))
    def _writeback0():
        pltpu.sync_copy(cnt_sh, cnt0_hbm)

    @pl.when((s == 0) & (c == 1))
    def _writeback1():
        pltpu.sync_copy(cnt_sh, cnt1_hbm)


def _count_incoming(tgt):
    zeros = jnp.zeros((N_NODES,), jnp.float32)
    ones = jnp.ones((EDGES_PER_WORKER,), jnp.float32)
    mesh = plsc.VectorSubcoreMesh(
        core_axis_name="c", subcore_axis_name="s",
        num_cores=NUM_SC, num_subcores=NUM_SUBCORES)
    f = pl.kernel(
        _sc_count_body,
        out_type=(jax.ShapeDtypeStruct((N_NODES,), jnp.float32),
                  jax.ShapeDtypeStruct((N_NODES,), jnp.float32)),
        mesh=mesh,
        scratch_types=[
            pltpu.VMEM((EDGES_PER_WORKER,), jnp.int32),
            pltpu.VMEM((EDGES_PER_WORKER,), jnp.float32),
            pltpu.VMEM_SHARED((N_NODES,), jnp.float32),
        ],
    )
    return f(tgt, zeros, ones)


def _proj_body(x_ref, wvt_ref, wot_ref, o_ref):
    t = jnp.dot(x_ref[...], wvt_ref[...], preferred_element_type=jnp.float32)
    o_ref[...] = jnp.dot(t, wot_ref[...], preferred_element_type=jnp.float32)


def _projection(x, wvt, wot):
    return pl.pallas_call(
        _proj_body,
        grid=(N_NODES // ROW_BLOCK,),
        in_specs=[
            pl.BlockSpec((ROW_BLOCK, HIDDEN), lambda i: (i, 0)),
            pl.BlockSpec((HIDDEN, HIDDEN), lambda i: (0, 0)),
            pl.BlockSpec((HIDDEN, HIDDEN), lambda i: (0, 0)),
        ],
        out_specs=pl.BlockSpec((ROW_BLOCK, HIDDEN), lambda i: (i, 0)),
        out_shape=jax.ShapeDtypeStruct((N_NODES, HIDDEN), jnp.float32),
    )(x, wvt, wot)


def _apply_body(y_ref, c0_ref, c1_ref, b_ref, o_ref):
    total = c0_ref[...] + c1_ref[...]
    o_ref[...] = jnp.where(total > 0.0, y_ref[...], 0.0) + b_ref[...]


def _mask_apply(y, c0, c1, b2d):
    return pl.pallas_call(
        _apply_body,
        grid=(N_NODES // ROW_BLOCK,),
        in_specs=[
            pl.BlockSpec((ROW_BLOCK, HIDDEN), lambda i: (i, 0)),
            pl.BlockSpec((ROW_BLOCK, 1), lambda i: (i, 0)),
            pl.BlockSpec((ROW_BLOCK, 1), lambda i: (i, 0)),
            pl.BlockSpec((1, HIDDEN), lambda i: (0, 0)),
        ],
        out_specs=pl.BlockSpec((ROW_BLOCK, HIDDEN), lambda i: (i, 0)),
        out_shape=jax.ShapeDtypeStruct((N_NODES, HIDDEN), jnp.float32),
    )(y, c0, c1, b2d)


def kernel(x, edge_index, edge_attr, W_q, W_k, W_v, W_edge, W_out, b_out):
    c0, c1 = _count_incoming(edge_index[1].astype(jnp.int32))
    y = _projection(x, W_v.T, W_out.T)
    return _mask_apply(y, c0.reshape(N_NODES, 1), c1.reshape(N_NODES, 1),
                       b_out.reshape(1, HIDDEN))


# speedup vs baseline: 1.0283x; 1.0283x over previous
import jax
import jax.numpy as jnp
from jax import lax
from jax.experimental import pallas as pl
from jax.experimental.pallas import tpu as pltpu
from jax.experimental.pallas import tpu_sc as plsc

N_NODES = 10000
N_EDGES = 320000
HIDDEN = 128
NUM_SC = 2
NUM_SUBCORES = 16
NUM_WORKERS = NUM_SC * NUM_SUBCORES
EDGES_PER_WORKER = N_EDGES // NUM_WORKERS
ROW_BLOCK = 1000


def _sc_count_body(tgt_hbm, zeros_hbm, ones_hbm, cnt0_hbm, cnt1_hbm,
                   idx_v, ones_v, cnt_sh):
    c = lax.axis_index("c")
    s = lax.axis_index("s")
    wid = c * NUM_SUBCORES + s

    @pl.when(s == 0)
    def _zero():
        pltpu.sync_copy(zeros_hbm, cnt_sh)

    plsc.subcore_barrier()

    pltpu.sync_copy(
        tgt_hbm.at[pl.ds(wid * EDGES_PER_WORKER, EDGES_PER_WORKER)], idx_v)
    pltpu.sync_copy(ones_hbm, ones_v)
    pltpu.sync_copy(ones_v, cnt_sh.at[idx_v], add=True)

    plsc.subcore_barrier()

    @pl.when((s == 0) & (c == 0))
    def _writeback0():
        pltpu.sync_copy(cnt_sh, cnt0_hbm)

    @pl.when((s == 0) & (c == 1))
    def _writeback1():
        pltpu.sync_copy(cnt_sh, cnt1_hbm)


def _count_incoming(tgt):
    zeros = jnp.zeros((N_NODES,), jnp.float32)
    ones = jnp.ones((EDGES_PER_WORKER,), jnp.float32)
    mesh = plsc.VectorSubcoreMesh(
        core_axis_name="c", subcore_axis_name="s",
        num_cores=NUM_SC, num_subcores=NUM_SUBCORES)
    f = pl.kernel(
        _sc_count_body,
        out_type=(jax.ShapeDtypeStruct((N_NODES,), jnp.float32),
                  jax.ShapeDtypeStruct((N_NODES,), jnp.float32)),
        mesh=mesh,
        scratch_types=[
            pltpu.VMEM((EDGES_PER_WORKER,), jnp.int32),
            pltpu.VMEM((EDGES_PER_WORKER,), jnp.float32),
            pltpu.VMEM_SHARED((N_NODES,), jnp.float32),
        ],
    )
    return f(tgt, zeros, ones)


def _tc_body(x_ref, wvt_ref, wot_ref, b_ref, c0_ref, c1_ref, o_ref):
    t = jnp.dot(x_ref[...], wvt_ref[...], preferred_element_type=jnp.float32)
    y = jnp.dot(t, wot_ref[...], preferred_element_type=jnp.float32)
    total = c0_ref[...] + c1_ref[...]
    o_ref[...] = jnp.where(total > 0.0, y, 0.0) + b_ref[...]


def _masked_projection(x, wvt, wot, b2d, c0, c1):
    return pl.pallas_call(
        _tc_body,
        grid=(N_NODES // ROW_BLOCK,),
        in_specs=[
            pl.BlockSpec((ROW_BLOCK, HIDDEN), lambda i: (i, 0)),
            pl.BlockSpec((HIDDEN, HIDDEN), lambda i: (0, 0)),
            pl.BlockSpec((HIDDEN, HIDDEN), lambda i: (0, 0)),
            pl.BlockSpec((1, HIDDEN), lambda i: (0, 0)),
            pl.BlockSpec((ROW_BLOCK, 1), lambda i: (i, 0)),
            pl.BlockSpec((ROW_BLOCK, 1), lambda i: (i, 0)),
        ],
        out_specs=pl.BlockSpec((ROW_BLOCK, HIDDEN), lambda i: (i, 0)),
        out_shape=jax.ShapeDtypeStruct((N_NODES, HIDDEN), jnp.float32),
    )(x, wvt, wot, b2d, c0, c1)


def kernel(x, edge_index, edge_attr, W_q, W_k, W_v, W_edge, W_out, b_out):
    c0, c1 = _count_incoming(edge_index[1].astype(jnp.int32))
    return _masked_projection(
        x, W_v.T, W_out.T, b_out.reshape(1, HIDDEN),
        c0.reshape(N_NODES, 1), c1.reshape(N_NODES, 1))


# speedup vs baseline: 1.2376x; 1.2035x over previous
import jax
import jax.numpy as jnp
from jax import lax
from jax.experimental import pallas as pl
from jax.experimental.pallas import tpu as pltpu
from jax.experimental.pallas import tpu_sc as plsc

N_NODES = 10000
N_EDGES = 320000
HIDDEN = 128
NUM_SC = 2
NUM_SUBCORES = 16
NUM_WORKERS = NUM_SC * NUM_SUBCORES
EDGES_PER_WORKER = N_EDGES // NUM_WORKERS
ROW_BLOCK = 2000


def _sc_count_body(tgt_hbm, zeros_hbm, ones_hbm, cnt_hbm, idx_v, ones_v, cnt_sh):
    c = lax.axis_index("c")
    s = lax.axis_index("s")
    wid = c * NUM_SUBCORES + s

    @pl.when(s == 0)
    def _zero():
        pltpu.sync_copy(zeros_hbm, cnt_sh)

    plsc.subcore_barrier()

    pltpu.sync_copy(tgt_hbm.at[pl.ds(wid * EDGES_PER_WORKER, EDGES_PER_WORKER)], idx_v)
    pltpu.sync_copy(ones_hbm, ones_v)
    pltpu.sync_copy(ones_v, cnt_sh.at[idx_v], add=True)

    plsc.subcore_barrier()

    @pl.when(s == 0)
    def _writeback():
        pltpu.sync_copy(cnt_sh, cnt_hbm.at[c])


def _count_incoming(tgt):
    zeros = jnp.zeros((N_NODES,), jnp.float32)
    ones = jnp.ones((EDGES_PER_WORKER,), jnp.float32)
    mesh = plsc.VectorSubcoreMesh(
        core_axis_name="c", subcore_axis_name="s",
        num_cores=NUM_SC, num_subcores=NUM_SUBCORES)
    f = pl.kernel(
        _sc_count_body,
        out_type=jax.ShapeDtypeStruct((NUM_SC, N_NODES), jnp.float32),
        mesh=mesh,
        scratch_types=[
            pltpu.VMEM((EDGES_PER_WORKER,), jnp.int32),
            pltpu.VMEM((EDGES_PER_WORKER,), jnp.float32),
            pltpu.VMEM_SHARED((N_NODES,), jnp.float32),
        ],
    )
    return f(tgt, zeros, ones)


def _tc_body(cnt_ref, x_ref, wvt_ref, wot_ref, b_ref, o_ref):
    t = jnp.dot(x_ref[...], wvt_ref[...], preferred_element_type=jnp.float32)
    y = jnp.dot(t, wot_ref[...], preferred_element_type=jnp.float32)
    total = cnt_ref[..., 0:1] + cnt_ref[..., 1:2]
    o_ref[...] = jnp.where(total > 0.0, y, 0.0) + b_ref[...]


def _masked_projection(counts_t, x, wvt, wot, b2d):
    return pl.pallas_call(
        _tc_body,
        grid=(N_NODES // ROW_BLOCK,),
        in_specs=[
            pl.BlockSpec((ROW_BLOCK, NUM_SC), lambda i: (i, 0)),
            pl.BlockSpec((ROW_BLOCK, HIDDEN), lambda i: (i, 0)),
            pl.BlockSpec((HIDDEN, HIDDEN), lambda i: (0, 0)),
            pl.BlockSpec((HIDDEN, HIDDEN), lambda i: (0, 0)),
            pl.BlockSpec((1, HIDDEN), lambda i: (0, 0)),
        ],
        out_specs=pl.BlockSpec((ROW_BLOCK, HIDDEN), lambda i: (i, 0)),
        out_shape=jax.ShapeDtypeStruct((N_NODES, HIDDEN), jnp.float32),
    )(counts_t, x, wvt, wot, b2d)


def kernel(x, edge_index, edge_attr, W_q, W_k, W_v, W_edge, W_out, b_out):
    tgt = edge_index[1].astype(jnp.int32)
    counts = _count_incoming(tgt)
    return _masked_projection(
        counts.T, x, W_v.T, W_out.T, b_out.reshape(1, HIDDEN))


# speedup vs baseline: 1.2748x; 1.0300x over previous
import jax
import jax.numpy as jnp
from jax import lax
from jax.experimental import pallas as pl
from jax.experimental.pallas import tpu as pltpu
from jax.experimental.pallas import tpu_sc as plsc

N_NODES = 10000
N_EDGES = 320000
HIDDEN = 128
NUM_SC = 2
NUM_SUBCORES = 16
NUM_WORKERS = NUM_SC * NUM_SUBCORES
EDGES_PER_WORKER = N_EDGES // NUM_WORKERS
ROW_BLOCK = 5000


def _sc_count_body(tgt_hbm, zeros_hbm, ones_hbm, cnt_hbm, idx_v, ones_v, cnt_sh):
    c = lax.axis_index("c")
    s = lax.axis_index("s")
    wid = c * NUM_SUBCORES + s

    @pl.when(s == 0)
    def _zero():
        pltpu.sync_copy(zeros_hbm, cnt_sh)

    plsc.subcore_barrier()

    pltpu.sync_copy(tgt_hbm.at[pl.ds(wid * EDGES_PER_WORKER, EDGES_PER_WORKER)], idx_v)
    pltpu.sync_copy(ones_hbm, ones_v)
    pltpu.sync_copy(ones_v, cnt_sh.at[idx_v], add=True)

    plsc.subcore_barrier()

    @pl.when(s == 0)
    def _writeback():
        pltpu.sync_copy(cnt_sh, cnt_hbm.at[c])


def _count_incoming(tgt):
    zeros = jnp.zeros((N_NODES,), jnp.float32)
    ones = jnp.ones((EDGES_PER_WORKER,), jnp.float32)
    mesh = plsc.VectorSubcoreMesh(
        core_axis_name="c", subcore_axis_name="s",
        num_cores=NUM_SC, num_subcores=NUM_SUBCORES)
    f = pl.kernel(
        _sc_count_body,
        out_type=jax.ShapeDtypeStruct((NUM_SC, N_NODES), jnp.float32),
        mesh=mesh,
        scratch_types=[
            pltpu.VMEM((EDGES_PER_WORKER,), jnp.int32),
            pltpu.VMEM((EDGES_PER_WORKER,), jnp.float32),
            pltpu.VMEM_SHARED((N_NODES,), jnp.float32),
        ],
    )
    return f(tgt, zeros, ones)


def _tc_body(cnt_ref, x_ref, wvt_ref, wot_ref, b_ref, o_ref):
    t = jnp.dot(x_ref[...], wvt_ref[...], preferred_element_type=jnp.float32)
    y = jnp.dot(t, wot_ref[...], preferred_element_type=jnp.float32)
    total = cnt_ref[..., 0:1] + cnt_ref[..., 1:2]
    o_ref[...] = jnp.where(total > 0.0, y, 0.0) + b_ref[...]


def _masked_projection(counts_t, x, wvt, wot, b2d):
    return pl.pallas_call(
        _tc_body,
        grid=(N_NODES // ROW_BLOCK,),
        in_specs=[
            pl.BlockSpec((ROW_BLOCK, NUM_SC), lambda i: (i, 0)),
            pl.BlockSpec((ROW_BLOCK, HIDDEN), lambda i: (i, 0)),
            pl.BlockSpec((HIDDEN, HIDDEN), lambda i: (0, 0)),
            pl.BlockSpec((HIDDEN, HIDDEN), lambda i: (0, 0)),
            pl.BlockSpec((1, HIDDEN), lambda i: (0, 0)),
        ],
        out_specs=pl.BlockSpec((ROW_BLOCK, HIDDEN), lambda i: (i, 0)),
        out_shape=jax.ShapeDtypeStruct((N_NODES, HIDDEN), jnp.float32),
    )(counts_t, x, wvt, wot, b2d)


def kernel(x, edge_index, edge_attr, W_q, W_k, W_v, W_edge, W_out, b_out):
    tgt = edge_index[1].astype(jnp.int32)
    counts = _count_incoming(tgt)
    return _masked_projection(
        counts.T, x, W_v.T, W_out.T, b_out.reshape(1, HIDDEN))
